# Initial kernel scaffold; baseline (speedup 1.0000x reference)
import jax
import jax.numpy as jnp
from jax.experimental import pallas as pl


def kernel(P, sl):
    raise NotImplementedError("write your pallas kernel here")


# speedup vs baseline: 4.7454x; 4.7454x over previous
import functools

import jax
import jax.numpy as jnp
from jax import lax
from jax.experimental import pallas as pl
from jax.experimental.pallas import tpu as pltpu
from jax.experimental.pallas import tpu_sc as plsc

NC = 2
NS = 16
L = 16
NW = NC * NS

TOTAL = 32768
SEGS = 16
SEG_LEN = TOTAL // SEGS
CHUNK = TOTAL // NW
VREGS = CHUNK // L
SEGS_PER_CORE = SEGS // NC

LN2 = 0.6931471805599453
SQRT2 = 1.4142135623730951


def _softlog(x):
    bits = lax.bitcast_convert_type(x, jnp.int32)
    e = lax.shift_right_logical(bits, 23) - 127
    m = lax.bitcast_convert_type((bits & 0x7FFFFF) | 0x3F800000, jnp.float32)
    big = m > SQRT2
    m = jnp.where(big, m * 0.5, m)
    e = jnp.where(big, e + 1, e)
    t = (m - 1.0) / (m + 1.0)
    t2 = t * t
    p = 2.0 / 7.0
    p = 2.0 / 5.0 + t2 * p
    p = 2.0 / 3.0 + t2 * p
    p = 2.0 + t2 * p
    return e.astype(jnp.float32) * LN2 + t * p


@functools.partial(
    pl.kernel,
    out_type=jax.ShapeDtypeStruct((SEGS,), jnp.float32),
    mesh=plsc.VectorSubcoreMesh(core_axis_name="c", subcore_axis_name="s"),
    compiler_params=pltpu.CompilerParams(needs_layout_passes=False),
    scratch_types=[
        pltpu.VMEM((CHUNK,), jnp.float32),
        pltpu.VMEM((L,), jnp.float32),
        pltpu.VMEM_SHARED((NS, L), jnp.float32),
        pltpu.VMEM((NS, L), jnp.float32),
        pltpu.VMEM((L,), jnp.float32),
        pltpu.VMEM((L,), jnp.float32),
    ],
)
def _seq_ll_sc(p_hbm, slf_hbm, out_hbm, chunk_v, part_v, shared, all_v,
               sl_v, out_v):
    c = lax.axis_index("c")
    s = lax.axis_index("s")
    h = c * NS + s

    pltpu.sync_copy(p_hbm.at[pl.ds(h * CHUNK, CHUNK)], chunk_v)

    def body(j, acc):
        return acc + _softlog(chunk_v[pl.ds(j * L, L)])

    acc = lax.fori_loop(0, VREGS, body, jnp.zeros((L,), jnp.float32))

    part_v[...] = acc
    pltpu.sync_copy(part_v, shared.at[s])
    plsc.subcore_barrier()

    @pl.when(s == 0)
    def _():
        pltpu.sync_copy(shared, all_v)
        pltpu.sync_copy(slf_hbm.at[pl.ds(c * SEGS_PER_CORE, SEGS_PER_CORE)],
                        sl_v.at[pl.ds(0, SEGS_PER_CORE)])
        lane = lax.iota(jnp.int32, L)
        out = jnp.zeros((L,), jnp.float32)
        for j in range(SEGS_PER_CORE):
            row = all_v[2 * j] + all_v[2 * j + 1]
            tot = jnp.sum(row)
            out = jnp.where(lane == j, tot, out)
        counts = jnp.where(lane < SEGS_PER_CORE, sl_v[...], 1.0)
        out_v[...] = -(out / counts)
        pltpu.sync_copy(out_v.at[pl.ds(0, SEGS_PER_CORE)],
                        out_hbm.at[pl.ds(c * SEGS_PER_CORE, SEGS_PER_CORE)])


def kernel(P, sl):
    return _seq_ll_sc(P, sl.astype(jnp.float32))
